# baseline (speedup 1.0000x reference)
import functools

import jax
import jax.numpy as jnp
from jax import lax
from jax.experimental import pallas as pl
from jax.experimental.pallas import tpu as pltpu
from jax.experimental.pallas import tpu_sc as plsc

BATCH = 16384
TOKENS = 50
DIM = 64
LANES = 16
VPR = DIM // LANES
ROWS = BATCH * TOKENS
NW = 32
G_ROWS = 100
G_PAD = 104
G_PER_CHUNK = 8
CHUNK_ROWS = G_ROWS * G_PER_CHUNK
NUM_CHUNKS = ROWS // CHUNK_ROWS
CHUNKS_PER_W = NUM_CHUNKS // NW

_mesh = plsc.VectorSubcoreMesh(core_axis_name="c", subcore_axis_name="s")


@functools.partial(
    pl.kernel,
    mesh=_mesh,
    out_type=jax.ShapeDtypeStruct(
        (NUM_CHUNKS, G_PER_CHUNK, G_ROWS, DIM), jnp.float32),
    scratch_types=[
        pltpu.VMEM((2, G_PER_CHUNK, G_PAD), jnp.int32),
        pltpu.VMEM((2, G_PER_CHUNK, G_PAD, DIM), jnp.float32),
        pltpu.VMEM((TOKENS, DIM), jnp.float32),
        pltpu.SemaphoreType.DMA,
        pltpu.SemaphoreType.DMA,
        pltpu.SemaphoreType.DMA,
        pltpu.SemaphoreType.DMA,
    ],
    compiler_params=pltpu.CompilerParams(use_tc_tiling_on_sc=False),
)
def _emb_kernel(idx_hbm, table_hbm, pos_hbm, out_hbm,
                idx_v, rows_v, pos_v, gsem0, gsem1, osem0, osem1):
    gsem = (gsem0, gsem1)
    osem = (osem0, osem1)
    wid = lax.axis_index("s") * 2 + lax.axis_index("c")
    base = wid * CHUNKS_PER_W
    pltpu.sync_copy(pos_hbm, pos_v)

    def start(b, c):
        pltpu.sync_copy(idx_hbm.at[base + c], idx_v.at[b])
        for j in range(G_PER_CHUNK):
            pltpu.async_copy(table_hbm.at[idx_v.at[b, j]],
                             rows_v.at[b, j], gsem[b])

    def drain_gathers(b):
        for j in range(G_PER_CHUNK):
            pltpu.make_async_copy(table_hbm.at[idx_v.at[b, j]],
                                  rows_v.at[b, j], gsem[b]).wait()

    def drain_outs(b):
        for j in range(G_PER_CHUNK):
            pltpu.make_async_copy(rows_v.at[b, j, pl.ds(0, G_ROWS)],
                                  out_hbm.at[0, j], osem[b]).wait()

    def finish(b, c):
        drain_gathers(b)

        def add_body(r, carry):
            pv = [pos_v[r, pl.ds(k * LANES, LANES)] for k in range(VPR)]
            for j in range(G_PER_CHUNK):
                for half in range(G_ROWS // TOKENS):
                    row = half * TOKENS + r
                    for k in range(VPR):
                        sl = pl.ds(k * LANES, LANES)
                        rows_v[b, j, row, sl] = rows_v[b, j, row, sl] + pv[k]
            return carry

        lax.fori_loop(0, TOKENS, add_body, 0)
        for j in range(G_PER_CHUNK):
            pltpu.async_copy(rows_v.at[b, j, pl.ds(0, G_ROWS)],
                             out_hbm.at[base + c, j], osem[b])

    start(0, 0)

    def body(i, carry):
        @pl.when(i > 0)
        def _():
            drain_outs(1)

        start(1, 2 * i + 1)
        finish(0, 2 * i)

        @pl.when(i < CHUNKS_PER_W // 2 - 1)
        def _():
            drain_outs(0)
            start(0, 2 * i + 2)

        finish(1, 2 * i + 1)
        return carry

    lax.fori_loop(0, CHUNKS_PER_W // 2, body, 0)
    drain_outs(0)
    drain_outs(1)


def kernel(observations, embedding_weight, position_code):
    idx = observations.astype(jnp.int32).reshape(
        NUM_CHUNKS, G_PER_CHUNK, G_ROWS)
    idx = jnp.pad(idx, ((0, 0), (0, 0), (0, G_PAD - G_ROWS)))
    pos = position_code.reshape(TOKENS, DIM)
    out = _emb_kernel(idx, embedding_weight, pos)
    return out.reshape(BATCH, TOKENS, DIM)


# speedup vs baseline: 1.0013x; 1.0013x over previous
import functools

import jax
import jax.numpy as jnp
from jax import lax
from jax.experimental import pallas as pl
from jax.experimental.pallas import tpu as pltpu
from jax.experimental.pallas import tpu_sc as plsc

BATCH = 16384
TOKENS = 50
DIM = 64
LANES = 16
VPR = DIM // LANES
ROWS = BATCH * TOKENS
NW = 32
G_ROWS = 100
G_PAD = 104
G_PER_CHUNK = 8
CHUNK_ROWS = G_ROWS * G_PER_CHUNK
CHUNK_PAD = G_PAD * G_PER_CHUNK
NUM_CHUNKS = ROWS // CHUNK_ROWS
CHUNKS_PER_W = NUM_CHUNKS // NW

_mesh = plsc.VectorSubcoreMesh(core_axis_name="c", subcore_axis_name="s")


@functools.partial(
    pl.kernel,
    mesh=_mesh,
    out_type=jax.ShapeDtypeStruct(
        (NUM_CHUNKS, G_PER_CHUNK, G_ROWS, DIM), jnp.float32),
    scratch_types=[
        pltpu.VMEM((2, CHUNK_PAD), jnp.int32),
        pltpu.VMEM((2, CHUNK_PAD, DIM), jnp.float32),
        pltpu.VMEM((TOKENS, DIM), jnp.float32),
        pltpu.SemaphoreType.DMA,
        pltpu.SemaphoreType.DMA,
        pltpu.SemaphoreType.DMA,
        pltpu.SemaphoreType.DMA,
    ],
    compiler_params=pltpu.CompilerParams(use_tc_tiling_on_sc=False),
)
def _emb_kernel(idx_hbm, table_hbm, pos_hbm, out_hbm,
                idx_v, rows_v, pos_v, gsem0, gsem1, osem0, osem1):
    gsem = (gsem0, gsem1)
    osem = (osem0, osem1)
    wid = lax.axis_index("s") * 2 + lax.axis_index("c")
    base = wid * CHUNKS_PER_W
    pltpu.sync_copy(pos_hbm, pos_v)

    def start(b, c):
        pltpu.sync_copy(idx_hbm.at[base + c], idx_v.at[b])
        pltpu.async_copy(table_hbm.at[idx_v.at[b]], rows_v.at[b], gsem[b])

    def drain_gathers(b):
        pltpu.make_async_copy(table_hbm.at[idx_v.at[b]],
                              rows_v.at[b], gsem[b]).wait()

    def drain_outs(b):
        for j in range(G_PER_CHUNK):
            pltpu.make_async_copy(rows_v.at[b, pl.ds(j * G_PAD, G_ROWS)],
                                  out_hbm.at[0, j], osem[b]).wait()

    def finish(b, c):
        drain_gathers(b)

        def add_body(r, carry):
            pv = [pos_v[r, pl.ds(k * LANES, LANES)] for k in range(VPR)]
            for j in range(G_PER_CHUNK):
                for half in range(G_ROWS // TOKENS):
                    row = j * G_PAD + half * TOKENS + r
                    for k in range(VPR):
                        sl = pl.ds(k * LANES, LANES)
                        rows_v[b, row, sl] = rows_v[b, row, sl] + pv[k]
            return carry

        lax.fori_loop(0, TOKENS, add_body, 0)
        for j in range(G_PER_CHUNK):
            pltpu.async_copy(rows_v.at[b, pl.ds(j * G_PAD, G_ROWS)],
                             out_hbm.at[base + c, j], osem[b])

    start(0, 0)

    def body(i, carry):
        @pl.when(i > 0)
        def _():
            drain_outs(1)

        start(1, 2 * i + 1)
        finish(0, 2 * i)

        @pl.when(i < CHUNKS_PER_W // 2 - 1)
        def _():
            drain_outs(0)
            start(0, 2 * i + 2)

        finish(1, 2 * i + 1)
        return carry

    lax.fori_loop(0, CHUNKS_PER_W // 2, body, 0)
    drain_outs(0)
    drain_outs(1)


def kernel(observations, embedding_weight, position_code):
    idx = observations.astype(jnp.int32).reshape(
        NUM_CHUNKS, G_PER_CHUNK, G_ROWS)
    idx = jnp.pad(idx, ((0, 0), (0, 0), (0, G_PAD - G_ROWS)))
    idx = idx.reshape(NUM_CHUNKS, CHUNK_PAD)
    pos = position_code.reshape(TOKENS, DIM)
    out = _emb_kernel(idx, embedding_weight, pos)
    return out.reshape(BATCH, TOKENS, DIM)
